# baseline (speedup 1.0000x reference)
import functools

import jax
import jax.numpy as jnp
from jax import lax
from jax.experimental import pallas as pl
from jax.experimental.pallas import tpu as pltpu, tpu_sc as plsc

NC = 2
NS = 16
L = 16
NW = NC * NS

B = 16384
D = 128
BPW = B // NW
CHUNK = 128
NCHUNK = BPW // CHUNK


def _mf_body(users_hbm, items_hbm, uf_hbm, if_hbm, ub_hbm, ib_hbm,
             out_hbm,
             idx_u, idx_i, u_buf, v_buf, ub_v, ib_v, out_v, sem):
    wid = lax.axis_index("s") * NC + lax.axis_index("c")
    base_row = pl.multiple_of(wid * NCHUNK, NCHUNK)
    base = pl.multiple_of(wid * BPW, BPW)

    pltpu.sync_copy(users_hbm.at[pl.ds(base_row, NCHUNK)], idx_u)
    pltpu.sync_copy(items_hbm.at[pl.ds(base_row, NCHUNK)], idx_i)

    lane = lax.iota(jnp.int32, L)

    for c in range(NCHUNK):
        d1 = pltpu.async_copy(uf_hbm.at[idx_u.at[c]], u_buf, sem)
        d2 = pltpu.async_copy(if_hbm.at[idx_i.at[c]], v_buf, sem)
        d3 = pltpu.async_copy(ub_hbm.at[idx_u.at[c]],
                              ub_v.at[pl.ds(c * CHUNK, CHUNK)], sem)
        d4 = pltpu.async_copy(ib_hbm.at[idx_i.at[c]],
                              ib_v.at[pl.ds(c * CHUNK, CHUNK)], sem)
        d1.wait()
        d2.wait()
        d3.wait()
        d4.wait()

        for g in range(CHUNK // L):
            rows = g * L + lane

            def d_body(i, acc, rows=rows):
                for k in range(4):
                    dd = jnp.full((L,), 0, jnp.int32) + (i * 4 + k)
                    u_d = plsc.load_gather(u_buf, [rows, dd])
                    v_d = plsc.load_gather(v_buf, [rows, dd])
                    acc = acc + u_d * v_d
                return acc

            acc = lax.fori_loop(0, D // 4, d_body,
                                jnp.zeros((L,), jnp.float32))
            off = c * CHUNK + g * L
            out_v[pl.ds(off, L)] = (acc + ub_v[pl.ds(off, L)]
                                    + ib_v[pl.ds(off, L)])

    pltpu.sync_copy(out_v, out_hbm.at[pl.ds(base, BPW)])


@functools.partial(jax.jit, static_argnames=())
def kernel(users, items, user_factors, item_factors, user_bias, item_bias,
           global_bias):
    mesh = plsc.VectorSubcoreMesh(core_axis_name="c", subcore_axis_name="s")
    run = pl.kernel(
        _mf_body,
        out_type=jax.ShapeDtypeStruct((B,), jnp.float32),
        mesh=mesh,
        compiler_params=pltpu.CompilerParams(needs_layout_passes=False),
        scratch_types=[
            pltpu.VMEM((NCHUNK, CHUNK), jnp.int32),
            pltpu.VMEM((NCHUNK, CHUNK), jnp.int32),
            pltpu.VMEM((CHUNK, D), jnp.float32),
            pltpu.VMEM((CHUNK, D), jnp.float32),
            pltpu.VMEM((BPW,), jnp.float32),
            pltpu.VMEM((BPW,), jnp.float32),
            pltpu.VMEM((BPW,), jnp.float32),
            pltpu.SemaphoreType.DMA,
        ],
    )
    users2d = users.reshape(B // CHUNK, CHUNK)
    items2d = items.reshape(B // CHUNK, CHUNK)
    ub = user_bias.reshape(-1)
    ib = item_bias.reshape(-1)
    out = run(users2d, items2d, user_factors, item_factors, ub, ib)
    return out + global_bias[0]


# speedup vs baseline: 1.0329x; 1.0329x over previous
import functools

import jax
import jax.numpy as jnp
from jax import lax
from jax.experimental import pallas as pl
from jax.experimental.pallas import tpu as pltpu, tpu_sc as plsc

NC = 2
NS = 16
L = 16
NW = NC * NS

B = 16384
D = 128
BPW = B // NW
CHUNK = 128
NCHUNK = BPW // CHUNK
NBUF = 3


def _mf_body(users_hbm, items_hbm, uf_hbm, if_hbm,
             ub_hbm, ib_hbm, out_hbm,
             fidx_u, fidx_i,
             u_bufs, v_bufs, ub_v, ib_v, out_v, sem_f, sem_b):
    wid = lax.axis_index("s") * NC + lax.axis_index("c")
    base = pl.multiple_of(wid * BPW, BPW)

    pltpu.sync_copy(users_hbm.at[pl.ds(base, BPW)], fidx_u)
    pltpu.sync_copy(items_hbm.at[pl.ds(base, BPW)], fidx_i)

    descs = {}

    def issue(c):
        slot = c % NBUF
        descs[(c, 0)] = pltpu.async_copy(
            uf_hbm.at[fidx_u.at[pl.ds(c * CHUNK, CHUNK)]], u_bufs[slot],
            sem_f)
        descs[(c, 1)] = pltpu.async_copy(
            if_hbm.at[fidx_i.at[pl.ds(c * CHUNK, CHUNK)]], v_bufs[slot],
            sem_f)

    for c in range(NBUF):
        issue(c)
    d_ub = pltpu.async_copy(ub_hbm.at[fidx_u], ub_v, sem_b)
    d_ib = pltpu.async_copy(ib_hbm.at[fidx_i], ib_v, sem_b)

    lane = lax.iota(jnp.int32, L)

    for c in range(NCHUNK):
        slot = c % NBUF
        descs[(c, 0)].wait()
        descs[(c, 1)].wait()
        u_buf = u_bufs[slot]
        v_buf = v_bufs[slot]

        for g in range(CHUNK // L):
            rows = g * L + lane

            def d_body(i, acc, rows=rows, u_buf=u_buf, v_buf=v_buf):
                for k in range(4):
                    dd = jnp.full((L,), 0, jnp.int32) + (i * 4 + k)
                    u_d = plsc.load_gather(u_buf, [rows, dd])
                    v_d = plsc.load_gather(v_buf, [rows, dd])
                    acc = acc + u_d * v_d
                return acc

            acc = lax.fori_loop(0, D // 4, d_body,
                                jnp.zeros((L,), jnp.float32))
            out_v[pl.ds(c * CHUNK + g * L, L)] = acc

        if c + NBUF < NCHUNK:
            issue(c + NBUF)

    d_ub.wait()
    d_ib.wait()
    for g in range(BPW // L):
        off = g * L
        out_v[pl.ds(off, L)] = (out_v[pl.ds(off, L)] + ub_v[pl.ds(off, L)]
                                + ib_v[pl.ds(off, L)])

    pltpu.sync_copy(out_v, out_hbm.at[pl.ds(base, BPW)])


@functools.partial(jax.jit, static_argnames=())
def kernel(users, items, user_factors, item_factors, user_bias, item_bias,
           global_bias):
    mesh = plsc.VectorSubcoreMesh(core_axis_name="c", subcore_axis_name="s")
    run = pl.kernel(
        _mf_body,
        out_type=jax.ShapeDtypeStruct((B,), jnp.float32),
        mesh=mesh,
        compiler_params=pltpu.CompilerParams(needs_layout_passes=False),
        scratch_types=[
            pltpu.VMEM((BPW,), jnp.int32),
            pltpu.VMEM((BPW,), jnp.int32),
            [pltpu.VMEM((CHUNK, D), jnp.float32)] * NBUF,
            [pltpu.VMEM((CHUNK, D), jnp.float32)] * NBUF,
            pltpu.VMEM((BPW,), jnp.float32),
            pltpu.VMEM((BPW,), jnp.float32),
            pltpu.VMEM((BPW,), jnp.float32),
            pltpu.SemaphoreType.DMA,
            pltpu.SemaphoreType.DMA,
        ],
    )
    ub = user_bias.reshape(-1)
    ib = item_bias.reshape(-1)
    out = run(users, items, user_factors, item_factors, ub, ib)
    return out + global_bias[0]
